# baseline (speedup 1.0000x reference)
import functools

import jax
import jax.numpy as jnp
from jax import lax
from jax.experimental import pallas as pl
from jax.experimental.pallas import tpu as pltpu
from jax.experimental.pallas import tpu_sc as plsc

_K = 512
_NC = 2
_NS = 16
_CG = 16
_CHUNK = 1024
_SCCHUNK = 512


def _thresh_body(x_ref, v_ref, key_ref):
    xb = x_ref[0]
    u = lax.bitcast_convert_type(xb, jnp.uint32)
    top = jnp.uint32(0x80000000)
    key_ref[...] = jnp.where(u >= top, ~u, u + top)

    l, c = key_ref.shape
    lo = jnp.zeros((1, c), jnp.uint32)
    for bit in range(31, -1, -1):
        t = lo | jnp.uint32(1 << bit)
        def count_chunk(i, acc):
            blk = key_ref[pl.ds(i * _CHUNK, _CHUNK), :]
            ge = (blk >= t).astype(jnp.int32)
            return acc + jnp.sum(ge, axis=0, keepdims=True)
        cnt = lax.fori_loop(0, l // _CHUNK, count_chunk,
                            jnp.zeros((1, c), jnp.int32))
        lo = jnp.where(cnt >= _K, t, lo)

    u2 = jnp.where(lo >= top, lo - top, ~lo)
    v = lax.bitcast_convert_type(u2, jnp.float32)
    v_ref[0] = jnp.broadcast_to(v, (8, c))


def _thresholds(x):
    b, l, c = x.shape
    return pl.pallas_call(
        _thresh_body,
        grid=(b,),
        in_specs=[pl.BlockSpec((1, l, c), lambda i: (i, 0, 0))],
        out_specs=pl.BlockSpec((1, 8, c), lambda i: (i, 0, 0)),
        out_shape=jax.ShapeDtypeStruct((b, 8, c), jnp.float32),
        scratch_shapes=[pltpu.VMEM((l, c), jnp.uint32)],
        compiler_params=pltpu.CompilerParams(
            dimension_semantics=("arbitrary",)
        ),
    )(x)


def _compact(x, v8):
    b, l, c = x.shape
    ng = c // _CG
    mesh = plsc.VectorSubcoreMesh(core_axis_name="c", subcore_axis_name="s")

    @functools.partial(
        pl.kernel,
        mesh=mesh,
        out_type=jax.ShapeDtypeStruct((b, 1, ng * _K * _CG), jnp.float32),
        compiler_params=pltpu.CompilerParams(needs_layout_passes=False),
        scratch_types=[
            pltpu.VMEM((_SCCHUNK, c), jnp.float32),
            pltpu.VMEM((ng * _K * _CG,), jnp.float32),
            pltpu.VMEM((8, c), jnp.float32),
        ],
    )
    def sc_kernel(x_hbm, v_hbm, out_hbm, xbuf, cand, vbuf):
        bi = lax.axis_index("s") * _NC + lax.axis_index("c")
        lanes = lax.iota(jnp.int32, _CG)
        pltpu.sync_copy(v_hbm.at[bi], vbuf)
        vvs = [vbuf[0, pl.ds(g * _CG, _CG)] for g in range(ng)]
        offs = [lanes + jnp.int32(g * _K * _CG) for g in range(ng)]

        def init_row(s, _):
            for g in range(ng):
                cand[pl.ds((g * _K + s) * _CG, _CG)] = vvs[g]
            return 0
        lax.fori_loop(0, _K, init_row, 0)

        def chunk_body(ci, cnts):
            pltpu.sync_copy(x_hbm.at[bi, pl.ds(ci * _SCCHUNK, _SCCHUNK)], xbuf)

            def row(i, cnts):
                new = []
                for g in range(ng):
                    xv = xbuf[i, pl.ds(g * _CG, _CG)]
                    m = xv > vvs[g]
                    idx = cnts[g] * _CG + offs[g]
                    plsc.store_scatter(cand, [idx], xv, mask=m)
                    new.append(cnts[g] + jnp.where(m, 1, 0))
                return tuple(new)
            return lax.fori_loop(0, _SCCHUNK, row, cnts)

        zero = jnp.zeros((_CG,), jnp.int32)
        lax.fori_loop(0, l // _SCCHUNK, chunk_body, (zero,) * ng)
        pltpu.sync_copy(cand, out_hbm.at[bi, 0])

    return sc_kernel(x, v8)


def _stage(x, j, k, desc):
    n, _ = x.shape
    i = lax.broadcasted_iota(jnp.int32, (n, 1), 0)
    bitj = (i & j) != 0
    up = pltpu.roll(x, n - j, 0)
    dn = pltpu.roll(x, j, 0)
    partner = jnp.where(bitj, dn, up)
    mx = jnp.maximum(x, partner)
    mn = jnp.minimum(x, partner)
    keepmax = bitj != ((i & k) != 0)
    if desc:
        keepmax = jnp.logical_not(keepmax)
    return jnp.where(keepmax, mx, mn)


def _sort_desc(x):
    n = x.shape[0]
    k = 2
    while k <= n:
        j = k // 2
        while j >= 1:
            x = _stage(x, j, k, True)
            j //= 2
        k *= 2
    return x


def _sort_body(c_ref, o_ref):
    xb = c_ref[...]
    s = jnp.concatenate([xb[0], xb[1]], axis=-1)
    s = _sort_desc(s)
    c = o_ref.shape[-1]
    o_ref[0] = s[:, :c]
    o_ref[1] = s[:, c:]


def _sort_candidates(cand):
    b, k, c = cand.shape
    return pl.pallas_call(
        _sort_body,
        grid=(b // 2,),
        in_specs=[pl.BlockSpec((2, k, c), lambda i: (i, 0, 0))],
        out_specs=pl.BlockSpec((2, k, c), lambda i: (i, 0, 0)),
        out_shape=jax.ShapeDtypeStruct((b, k, c), jnp.float32),
        compiler_params=pltpu.CompilerParams(
            dimension_semantics=("parallel",)
        ),
    )(cand)


def kernel(x):
    b, l, c = x.shape
    v8 = _thresholds(x)
    cand4 = _compact(x, v8).reshape(b, c // _CG, _K, _CG)
    cand = cand4.transpose(0, 2, 1, 3).reshape(b, _K, c)
    return _sort_candidates(cand)


# speedup vs baseline: 1.4094x; 1.4094x over previous
import functools

import jax
import jax.numpy as jnp
from jax import lax
from jax.experimental import pallas as pl
from jax.experimental.pallas import tpu as pltpu
from jax.experimental.pallas import tpu_sc as plsc

_K = 512
_NC = 2
_NS = 16
_CG = 16
_CHUNK = 1024
_SCCHUNK = 512


def _thresh_body(x_ref, v_ref, key_ref):
    li = pl.program_id(1)
    nl = pl.num_programs(1)
    xb = x_ref[...]
    xw = jnp.concatenate([xb[0], xb[1]], axis=-1)
    u = lax.bitcast_convert_type(xw, jnp.uint32)
    top = jnp.uint32(0x80000000)
    keyu = jnp.where(u >= top, ~u, u + top)
    lpart = xw.shape[0]
    key_ref[pl.ds(li * lpart, lpart), :] = lax.bitcast_convert_type(
        keyu ^ top, jnp.int32)

    @pl.when(li == nl - 1)
    def _():
        l, c2 = key_ref.shape
        nch = l // _CHUNK
        lo = jnp.zeros((1, c2), jnp.uint32)
        for bit in range(31, -1, -1):
            t = lo | jnp.uint32(1 << bit)
            ts = lax.bitcast_convert_type(t ^ top, jnp.int32)

            def count_chunk(i, acc8):
                blk = key_ref[pl.ds(i * _CHUNK, _CHUNK), :]
                ge = (blk >= ts).astype(jnp.int32)
                return acc8 + jnp.sum(ge.reshape(_CHUNK // 8, 8, c2), axis=0)
            acc8 = lax.fori_loop(0, nch, count_chunk,
                                 jnp.zeros((8, c2), jnp.int32))
            cnt = jnp.sum(acc8, axis=0, keepdims=True)
            lo = jnp.where(cnt >= _K, t, lo)

        u2 = jnp.where(lo >= top, lo - top, ~lo)
        v = lax.bitcast_convert_type(u2, jnp.float32)
        c = c2 // 2
        v_ref[0] = jnp.broadcast_to(v[:, :c], (8, c))
        v_ref[1] = jnp.broadcast_to(v[:, c:], (8, c))


def _thresholds(x):
    b, l, c = x.shape
    nl = 4
    return pl.pallas_call(
        _thresh_body,
        grid=(b // 2, nl),
        in_specs=[pl.BlockSpec((2, l // nl, c), lambda i, j: (i, j, 0))],
        out_specs=pl.BlockSpec((2, 8, c), lambda i, j: (i, 0, 0)),
        out_shape=jax.ShapeDtypeStruct((b, 8, c), jnp.float32),
        scratch_shapes=[pltpu.VMEM((l, 2 * c), jnp.int32)],
        compiler_params=pltpu.CompilerParams(
            dimension_semantics=("arbitrary", "arbitrary")
        ),
    )(x)


def _compact(x, v8):
    b, l, c = x.shape
    ng = c // _CG
    mesh = plsc.VectorSubcoreMesh(core_axis_name="c", subcore_axis_name="s")

    @functools.partial(
        pl.kernel,
        mesh=mesh,
        out_type=jax.ShapeDtypeStruct((b, 1, ng * _K * _CG), jnp.float32),
        compiler_params=pltpu.CompilerParams(needs_layout_passes=False),
        scratch_types=[
            pltpu.VMEM((_SCCHUNK, c), jnp.float32),
            pltpu.VMEM((ng * _K * _CG,), jnp.float32),
            pltpu.VMEM((8, c), jnp.float32),
        ],
    )
    def sc_kernel(x_hbm, v_hbm, out_hbm, xbuf, cand, vbuf):
        bi = lax.axis_index("s") * _NC + lax.axis_index("c")
        lanes = lax.iota(jnp.int32, _CG)
        pltpu.sync_copy(v_hbm.at[bi], vbuf)
        vvs = [vbuf[0, pl.ds(g * _CG, _CG)] for g in range(ng)]
        offs = [lanes + jnp.int32(g * _K * _CG) for g in range(ng)]

        def init_row(s, _):
            for g in range(ng):
                cand[pl.ds((g * _K + s) * _CG, _CG)] = vvs[g]
            return 0
        lax.fori_loop(0, _K, init_row, 0)

        def chunk_body(ci, cnts):
            pltpu.sync_copy(x_hbm.at[bi, pl.ds(ci * _SCCHUNK, _SCCHUNK)], xbuf)

            def row(i, cnts):
                new = []
                for g in range(ng):
                    xv = xbuf[i, pl.ds(g * _CG, _CG)]
                    m = xv > vvs[g]
                    idx = cnts[g] * _CG + offs[g]
                    plsc.store_scatter(cand, [idx], xv, mask=m)
                    new.append(cnts[g] + jnp.where(m, 1, 0))
                return tuple(new)
            return lax.fori_loop(0, _SCCHUNK, row, cnts)

        zero = jnp.zeros((_CG,), jnp.int32)
        lax.fori_loop(0, l // _SCCHUNK, chunk_body, (zero,) * ng)
        pltpu.sync_copy(cand, out_hbm.at[bi, 0])

    return sc_kernel(x, v8)


def _stage(x, j, k, desc):
    n, _ = x.shape
    i = lax.broadcasted_iota(jnp.int32, (n, 1), 0)
    bitj = (i & j) != 0
    up = pltpu.roll(x, n - j, 0)
    dn = pltpu.roll(x, j, 0)
    partner = jnp.where(bitj, dn, up)
    mx = jnp.maximum(x, partner)
    mn = jnp.minimum(x, partner)
    keepmax = bitj != ((i & k) != 0)
    if desc:
        keepmax = jnp.logical_not(keepmax)
    return jnp.where(keepmax, mx, mn)


def _sort_desc(x):
    n = x.shape[0]
    k = 2
    while k <= n:
        j = k // 2
        while j >= 1:
            x = _stage(x, j, k, True)
            j //= 2
        k *= 2
    return x


def _sort_body(c_ref, o_ref):
    xb = c_ref[...]
    s = jnp.concatenate([xb[0], xb[1]], axis=-1)
    s = _sort_desc(s)
    c = o_ref.shape[-1]
    o_ref[0] = s[:, :c]
    o_ref[1] = s[:, c:]


def _sort_candidates(cand):
    b, k, c = cand.shape
    return pl.pallas_call(
        _sort_body,
        grid=(b // 2,),
        in_specs=[pl.BlockSpec((2, k, c), lambda i: (i, 0, 0))],
        out_specs=pl.BlockSpec((2, k, c), lambda i: (i, 0, 0)),
        out_shape=jax.ShapeDtypeStruct((b, k, c), jnp.float32),
        compiler_params=pltpu.CompilerParams(
            dimension_semantics=("parallel",)
        ),
    )(cand)


def kernel(x):
    b, l, c = x.shape
    v8 = _thresholds(x)
    cand4 = _compact(x, v8).reshape(b, c // _CG, _K, _CG)
    cand = cand4.transpose(0, 2, 1, 3).reshape(b, _K, c)
    return _sort_candidates(cand)
